# Initial kernel scaffold; baseline (speedup 1.0000x reference)
import functools

import jax
import jax.numpy as jnp
from jax import lax
from jax.experimental import pallas as pl
from jax.experimental.pallas import tpu as pltpu
from jax.experimental.pallas import tpu_sc as plsc

NC = 2
NS = 16
NW = NC * NS
CH = 128
CW = 16
LANES = 16


def _round_up(v, m):
    return (v + m - 1) // m * m


@functools.lru_cache(maxsize=None)
def _make_sc_agg(n, d, cpt, nacc, with_counts):
    rpt = nacc // NS
    assert rpt % CH == 0
    mesh = plsc.VectorSubcoreMesh(core_axis_name="c", subcore_axis_name="s")

    out_type = [jax.ShapeDtypeStruct((NC, nacc, d), jnp.float32)]
    scratch = [
        pltpu.VMEM_SHARED((nacc, d), jnp.float32),
        pltpu.VMEM((cpt, CH), jnp.int32),
        pltpu.VMEM((cpt, CH), jnp.int32),
        pltpu.VMEM((CH, d), jnp.float32),
        pltpu.VMEM((CH, d), jnp.float32),
        pltpu.SemaphoreType.DMA,
        pltpu.SemaphoreType.DMA,
    ]
    if with_counts:
        out_type.append(jax.ShapeDtypeStruct((NC, nacc, CW), jnp.float32))
        scratch += [
            pltpu.VMEM_SHARED((nacc, CW), jnp.float32),
            pltpu.VMEM((CH, CW), jnp.float32),
        ]

    def body(h, srcs, dsts, *rest):
        if with_counts:
            part, cntp, acc, sidx, didx, bufA, bufB, semA, semB, cacc, ones = rest
        else:
            part, acc, sidx, didx, bufA, bufB, semA, semB = rest
            cntp = cacc = ones = None
        cid = lax.axis_index("c")
        sid = lax.axis_index("s")
        wid = cid * NS + sid

        pltpu.sync_copy(srcs.at[pl.ds(wid * cpt, cpt)], sidx)
        pltpu.sync_copy(dsts.at[pl.ds(wid * cpt, cpt)], didx)

        zv = jnp.zeros((LANES,), jnp.float32)
        for j in range(d // LANES):
            bufA[0, pl.ds(j * LANES, LANES)] = zv
        k = 1
        while k < CH:
            pltpu.sync_copy(bufA.at[pl.ds(0, k)], bufA.at[pl.ds(k, k)])
            k *= 2
        for r in range(rpt // CH):
            pltpu.sync_copy(bufA, acc.at[pl.ds(sid * rpt + r * CH, CH)])

        if with_counts:
            ones[0, pl.ds(0, LANES)] = zv
            k = 1
            while k < CH:
                pltpu.sync_copy(ones.at[pl.ds(0, k)], ones.at[pl.ds(k, k)])
                k *= 2
            for r in range(rpt // CH):
                pltpu.sync_copy(ones, cacc.at[pl.ds(sid * rpt + r * CH, CH)])
            ones[0, pl.ds(0, LANES)] = jnp.ones((LANES,), jnp.float32)
            k = 1
            while k < CH:
                pltpu.sync_copy(ones.at[pl.ds(0, k)], ones.at[pl.ds(k, k)])
                k *= 2

        plsc.subcore_barrier()

        bufs = (bufA, bufB)
        sems = (semA, semB)
        pltpu.async_copy(h.at[sidx.at[0]], bufA, semA)

        def outer(jj, carry):
            for b in range(2):
                j = jj * 2 + b
                pltpu.make_async_copy(h.at[sidx.at[j]], bufs[b], sems[b]).wait()

                @pl.when(j + 1 < cpt)
                def _():
                    pltpu.async_copy(h.at[sidx.at[j + 1]], bufs[1 - b],
                                     sems[1 - b])

                pltpu.sync_copy(bufs[b], acc.at[didx.at[j]], add=True)
                if with_counts:
                    pltpu.sync_copy(ones, cacc.at[didx.at[j]], add=True)
            return carry

        lax.fori_loop(0, cpt // 2, outer, 0)

        plsc.subcore_barrier()

        pltpu.sync_copy(acc.at[pl.ds(sid * rpt, rpt)],
                        part.at[cid, pl.ds(sid * rpt, rpt)])
        if with_counts:
            pltpu.sync_copy(cacc.at[pl.ds(sid * rpt, rpt)],
                            cntp.at[cid, pl.ds(sid * rpt, rpt)])

    return pl.kernel(body, out_type=out_type, mesh=mesh,
                     scratch_types=scratch)


def _pick_br(n):
    for b in (400, 500, 250, 200, 100, 40, 8):
        if n % b == 0:
            return b
    return n


def _combine1(mp, cp, x, wl, bl, wr):
    n, d = x.shape
    br = _pick_br(n)

    def body(mp_ref, cp_ref, x_ref, wl_ref, wr_ref, bl_ref, h_ref, inv_ref):
        cnt = jnp.sum(cp_ref[...], axis=(0, 2))
        inv = 1.0 / jnp.maximum(cnt, 1.0)
        inv2 = jnp.broadcast_to(inv[:, None], (br, d))
        m = (mp_ref[0] + mp_ref[1]) * inv2
        o = (jnp.dot(m, wl_ref[...], preferred_element_type=jnp.float32)
             + jnp.dot(x_ref[...], wr_ref[...],
                       preferred_element_type=jnp.float32)
             + bl_ref[...])
        h_ref[...] = jnp.maximum(o, 0.0)
        inv_ref[...] = inv2

    return pl.pallas_call(
        body,
        grid=(n // br,),
        in_specs=[
            pl.BlockSpec((NC, br, d), lambda i: (0, i, 0)),
            pl.BlockSpec((NC, br, CW), lambda i: (0, i, 0)),
            pl.BlockSpec((br, d), lambda i: (i, 0)),
            pl.BlockSpec((d, d), lambda i: (0, 0)),
            pl.BlockSpec((d, d), lambda i: (0, 0)),
            pl.BlockSpec((1, d), lambda i: (0, 0)),
        ],
        out_specs=[
            pl.BlockSpec((br, d), lambda i: (i, 0)),
            pl.BlockSpec((br, d), lambda i: (i, 0)),
        ],
        out_shape=[
            jax.ShapeDtypeStruct((n, d), jnp.float32),
            jax.ShapeDtypeStruct((n, d), jnp.float32),
        ],
    )(mp, cp, x, wl, wr, bl)


def _combine(mp, inv, x, wl, bl, wr, relu):
    n, d = x.shape
    br = _pick_br(n)

    def body(mp_ref, inv_ref, x_ref, wl_ref, wr_ref, bl_ref, h_ref):
        m = (mp_ref[0] + mp_ref[1]) * inv_ref[...]
        o = (jnp.dot(m, wl_ref[...], preferred_element_type=jnp.float32)
             + jnp.dot(x_ref[...], wr_ref[...],
                       preferred_element_type=jnp.float32)
             + bl_ref[...])
        if relu:
            o = jnp.maximum(o, 0.0)
        h_ref[...] = o

    return pl.pallas_call(
        body,
        grid=(n // br,),
        in_specs=[
            pl.BlockSpec((NC, br, d), lambda i: (0, i, 0)),
            pl.BlockSpec((br, d), lambda i: (i, 0)),
            pl.BlockSpec((br, d), lambda i: (i, 0)),
            pl.BlockSpec((d, d), lambda i: (0, 0)),
            pl.BlockSpec((d, d), lambda i: (0, 0)),
            pl.BlockSpec((1, d), lambda i: (0, 0)),
        ],
        out_specs=pl.BlockSpec((br, d), lambda i: (i, 0)),
        out_shape=jax.ShapeDtypeStruct((n, d), jnp.float32),
    )(mp, inv, x, wl, wr, bl)


def kernel(x, edge_index, Wl1, bl1, Wr1, Wl2, bl2, Wr2, Wl3, bl3, Wr3):
    n, d = x.shape
    e = edge_index.shape[1]
    cpt = _round_up(e, NW * CH) // (NW * CH)
    epad = NW * cpt * CH
    nacc = _round_up(n + 1, NS * CH)

    src = edge_index[0].astype(jnp.int32)
    dst = edge_index[1].astype(jnp.int32)
    pad = epad - e
    srcs = jnp.concatenate([src, jnp.zeros((pad,), jnp.int32)])
    dsts = jnp.concatenate([dst, jnp.full((pad,), n, jnp.int32)])
    srcs = srcs.reshape(epad // CH, CH)
    dsts = dsts.reshape(epad // CH, CH)

    agg1 = _make_sc_agg(n, d, cpt, nacc, True)
    agg = _make_sc_agg(n, d, cpt, nacc, False)

    mp1, cp1 = agg1(x, srcs, dsts)
    h1, inv = _combine1(mp1, cp1, x, Wl1, bl1.reshape(1, d), Wr1)
    mp2 = agg(h1, srcs, dsts)
    h2 = _combine(mp2, inv, h1, Wl2, bl2.reshape(1, d), Wr2, relu=True)
    mp3 = agg(h2, srcs, dsts)
    h3 = _combine(mp3, inv, h2, Wl3, bl3.reshape(1, d), Wr3, relu=False)
    return h3


# speedup vs baseline: 3.3703x; 3.3703x over previous
import functools

import jax
import jax.numpy as jnp
from jax import lax
from jax.experimental import pallas as pl
from jax.experimental.pallas import tpu as pltpu
from jax.experimental.pallas import tpu_sc as plsc

NC = 2
NS = 16
NW = NC * NS
CH = 128
CW = 16
LANES = 16


def _round_up(v, m):
    return (v + m - 1) // m * m


GPC = 8


@functools.lru_cache(maxsize=None)
def _make_sc_agg(n, d, cpt, nacc, with_counts):
    rpt = nacc // NS
    assert cpt % GPC == 0 and GPC % 2 == 0
    ngrp = cpt // GPC
    mesh = plsc.VectorSubcoreMesh(core_axis_name="c", subcore_axis_name="s")

    out_type = [jax.ShapeDtypeStruct((NC, nacc, d), jnp.float32)]
    scratch = [
        pltpu.VMEM_SHARED((nacc, d), jnp.float32),
        pltpu.VMEM((2, GPC, CH), jnp.int32),
        pltpu.VMEM((2, GPC, CH), jnp.int32),
        pltpu.VMEM((CH, d), jnp.float32),
        pltpu.VMEM((CH, d), jnp.float32),
        pltpu.SemaphoreType.DMA,
        pltpu.SemaphoreType.DMA,
        pltpu.SemaphoreType.DMA,
    ]
    if with_counts:
        out_type.append(jax.ShapeDtypeStruct((NC, nacc, CW), jnp.float32))
        scratch += [
            pltpu.VMEM_SHARED((nacc, CW), jnp.float32),
            pltpu.VMEM((CH, CW), jnp.float32),
        ]

    def body(h, srcs, dsts, *rest):
        if with_counts:
            (part, cntp, acc, sidx, didx, bufA, bufB,
             semA, semB, semI, cacc, ones) = rest
        else:
            part, acc, sidx, didx, bufA, bufB, semA, semB, semI = rest
            cntp = cacc = ones = None
        cid = lax.axis_index("c")
        sid = lax.axis_index("s")
        wid = cid * NS + sid
        base = wid * cpt

        zv = jnp.zeros((LANES,), jnp.float32)

        def zrow(i, c):
            for j in range(d // LANES):
                bufA[i, pl.ds(j * LANES, LANES)] = zv
            if with_counts:
                ones[i, pl.ds(0, LANES)] = zv
            return c

        lax.fori_loop(0, CH, zrow, 0)

        def _stripe_copy(src_buf, dst_ref):
            r0 = sid * rpt
            for r in range(rpt // CH):
                pltpu.sync_copy(src_buf, dst_ref.at[pl.ds(r0 + r * CH, CH)])
            rem = rpt % CH
            if rem:
                pltpu.sync_copy(src_buf.at[pl.ds(0, rem)],
                                dst_ref.at[pl.ds(r0 + rpt - rem, rem)])

        _stripe_copy(bufA, acc)
        if with_counts:
            _stripe_copy(ones, cacc)
            ov = jnp.ones((LANES,), jnp.float32)

            def orow(i, c):
                ones[i, pl.ds(0, LANES)] = ov
                return c

            lax.fori_loop(0, CH, orow, 0)

        plsc.subcore_barrier()

        bufs = (bufA, bufB)
        sems = (semA, semB)
        pltpu.sync_copy(srcs.at[pl.ds(base, GPC)], sidx.at[0])
        pltpu.sync_copy(dsts.at[pl.ds(base, GPC)], didx.at[0])
        pltpu.async_copy(h.at[sidx.at[0, 0]], bufA, semA)

        def group(g, carry):
            s = lax.rem(g, 2)
            sn = 1 - s

            @pl.when(g + 1 < ngrp)
            def _():
                off = base + (g + 1) * GPC
                pltpu.async_copy(srcs.at[pl.ds(off, GPC)], sidx.at[sn], semI)
                pltpu.async_copy(dsts.at[pl.ds(off, GPC)], didx.at[sn], semI)

            for b8 in range(GPC):
                b = b8 % 2
                pltpu.make_async_copy(h.at[sidx.at[s, b8]], bufs[b],
                                      sems[b]).wait()
                if b8 + 1 < GPC:
                    pltpu.async_copy(h.at[sidx.at[s, b8 + 1]], bufs[1 - b],
                                     sems[1 - b])
                else:
                    @pl.when(g + 1 < ngrp)
                    def _():
                        pltpu.make_async_copy(srcs.at[pl.ds(base, GPC)],
                                              sidx.at[sn], semI).wait()
                        pltpu.make_async_copy(dsts.at[pl.ds(base, GPC)],
                                              didx.at[sn], semI).wait()
                        pltpu.async_copy(h.at[sidx.at[sn, 0]], bufs[1 - b],
                                         sems[1 - b])
                pltpu.sync_copy(bufs[b], acc.at[didx.at[s, b8]], add=True)
                if with_counts:
                    pltpu.sync_copy(ones, cacc.at[didx.at[s, b8]], add=True)
            return carry

        lax.fori_loop(0, ngrp, group, 0)

        plsc.subcore_barrier()

        pltpu.sync_copy(acc.at[pl.ds(sid * rpt, rpt)],
                        part.at[cid, pl.ds(sid * rpt, rpt)])
        if with_counts:
            pltpu.sync_copy(cacc.at[pl.ds(sid * rpt, rpt)],
                            cntp.at[cid, pl.ds(sid * rpt, rpt)])

    return pl.kernel(body, out_type=out_type, mesh=mesh,
                     scratch_types=scratch,
                     compiler_params=pltpu.CompilerParams(
                         use_tc_tiling_on_sc=False))


def _pick_br(n):
    for b in (400, 500, 250, 200, 100, 40, 8):
        if n % b == 0:
            return b
    return n


def _combine1(mp, cp, x, wl, bl, wr):
    n, d = x.shape
    br = _pick_br(n)

    def body(mp_ref, cp_ref, x_ref, wl_ref, wr_ref, bl_ref, h_ref, inv_ref):
        cnt = jnp.sum(cp_ref[...], axis=(0, 2)) * (1.0 / CW)
        inv = 1.0 / jnp.maximum(cnt, 1.0)
        inv2 = jnp.broadcast_to(inv[:, None], (br, d))
        m = (mp_ref[0] + mp_ref[1]) * inv2
        o = (jnp.dot(m, wl_ref[...], preferred_element_type=jnp.float32)
             + jnp.dot(x_ref[...], wr_ref[...],
                       preferred_element_type=jnp.float32)
             + bl_ref[...])
        h_ref[...] = jnp.maximum(o, 0.0)
        inv_ref[...] = inv2

    return pl.pallas_call(
        body,
        grid=(n // br,),
        in_specs=[
            pl.BlockSpec((NC, br, d), lambda i: (0, i, 0)),
            pl.BlockSpec((NC, br, CW), lambda i: (0, i, 0)),
            pl.BlockSpec((br, d), lambda i: (i, 0)),
            pl.BlockSpec((d, d), lambda i: (0, 0)),
            pl.BlockSpec((d, d), lambda i: (0, 0)),
            pl.BlockSpec((1, d), lambda i: (0, 0)),
        ],
        out_specs=[
            pl.BlockSpec((br, d), lambda i: (i, 0)),
            pl.BlockSpec((br, d), lambda i: (i, 0)),
        ],
        out_shape=[
            jax.ShapeDtypeStruct((n, d), jnp.float32),
            jax.ShapeDtypeStruct((n, d), jnp.float32),
        ],
    )(mp, cp, x, wl, wr, bl)


def _combine(mp, inv, x, wl, bl, wr, relu):
    n, d = x.shape
    br = _pick_br(n)

    def body(mp_ref, inv_ref, x_ref, wl_ref, wr_ref, bl_ref, h_ref):
        m = (mp_ref[0] + mp_ref[1]) * inv_ref[...]
        o = (jnp.dot(m, wl_ref[...], preferred_element_type=jnp.float32)
             + jnp.dot(x_ref[...], wr_ref[...],
                       preferred_element_type=jnp.float32)
             + bl_ref[...])
        if relu:
            o = jnp.maximum(o, 0.0)
        h_ref[...] = o

    return pl.pallas_call(
        body,
        grid=(n // br,),
        in_specs=[
            pl.BlockSpec((NC, br, d), lambda i: (0, i, 0)),
            pl.BlockSpec((br, d), lambda i: (i, 0)),
            pl.BlockSpec((br, d), lambda i: (i, 0)),
            pl.BlockSpec((d, d), lambda i: (0, 0)),
            pl.BlockSpec((d, d), lambda i: (0, 0)),
            pl.BlockSpec((1, d), lambda i: (0, 0)),
        ],
        out_specs=pl.BlockSpec((br, d), lambda i: (i, 0)),
        out_shape=jax.ShapeDtypeStruct((n, d), jnp.float32),
    )(mp, inv, x, wl, wr, bl)


def kernel(x, edge_index, Wl1, bl1, Wr1, Wl2, bl2, Wr2, Wl3, bl3, Wr3):
    n, d = x.shape
    e = edge_index.shape[1]
    cpt = _round_up(_round_up(e, NW * CH) // (NW * CH), GPC)
    epad = NW * cpt * CH
    nacc = _round_up(n + 1, CH)

    src = edge_index[0].astype(jnp.int32)
    dst = edge_index[1].astype(jnp.int32)
    pad = epad - e
    srcs = jnp.concatenate([src, jnp.zeros((pad,), jnp.int32)])
    dsts = jnp.concatenate([dst, jnp.full((pad,), n, jnp.int32)])
    srcs = srcs.reshape(epad // CH, CH)
    dsts = dsts.reshape(epad // CH, CH)

    agg1 = _make_sc_agg(n, d, cpt, nacc, True)
    agg = _make_sc_agg(n, d, cpt, nacc, False)

    mp1, cp1 = agg1(x, srcs, dsts)
    h1, inv = _combine1(mp1, cp1, x, Wl1, bl1.reshape(1, d), Wr1)
    (mp2,) = agg(h1, srcs, dsts)
    h2 = _combine(mp2, inv, h1, Wl2, bl2.reshape(1, d), Wr2, relu=True)
    (mp3,) = agg(h2, srcs, dsts)
    h3 = _combine(mp3, inv, h2, Wl3, bl3.reshape(1, d), Wr3, relu=False)
    return h3


# speedup vs baseline: 3.3984x; 1.0083x over previous
import functools

import jax
import jax.numpy as jnp
from jax import lax
from jax.experimental import pallas as pl
from jax.experimental.pallas import tpu as pltpu
from jax.experimental.pallas import tpu_sc as plsc

NC = 2
NS = 16
NW = NC * NS
CH = 128
CW = 16
LANES = 16


def _round_up(v, m):
    return (v + m - 1) // m * m


GPC = 8


@functools.lru_cache(maxsize=None)
def _make_sc_agg(n, d, cpt, nacc, with_counts):
    rpt = nacc // NS
    assert cpt % GPC == 0 and GPC % 2 == 0
    ngrp = cpt // GPC
    mesh = plsc.VectorSubcoreMesh(core_axis_name="c", subcore_axis_name="s")

    out_type = [jax.ShapeDtypeStruct((NC, nacc, d), jnp.float32)]
    scratch = [
        pltpu.VMEM_SHARED((nacc, d), jnp.float32),
        pltpu.VMEM((2, GPC, CH), jnp.int32),
        pltpu.VMEM((2, GPC, CH), jnp.int32),
        pltpu.VMEM((CH, d), jnp.float32),
        pltpu.VMEM((CH, d), jnp.float32),
        pltpu.SemaphoreType.DMA,
        pltpu.SemaphoreType.DMA,
        pltpu.SemaphoreType.DMA,
        pltpu.SemaphoreType.DMA,
        pltpu.SemaphoreType.DMA,
    ]
    if with_counts:
        out_type.append(jax.ShapeDtypeStruct((NC, nacc, CW), jnp.float32))
        scratch += [
            pltpu.VMEM_SHARED((nacc, CW), jnp.float32),
            pltpu.VMEM((CH, CW), jnp.float32),
            pltpu.SemaphoreType.DMA,
        ]

    def body(h, srcs, dsts, *rest):
        if with_counts:
            (part, cntp, acc, sidx, didx, bufA, bufB,
             semA, semB, semSA, semSB, semI, cacc, ones, semC) = rest
        else:
            (part, acc, sidx, didx, bufA, bufB,
             semA, semB, semSA, semSB, semI) = rest
            cntp = cacc = ones = semC = None
        cid = lax.axis_index("c")
        sid = lax.axis_index("s")
        wid = cid * NS + sid
        base = wid * cpt

        zv = jnp.zeros((LANES,), jnp.float32)

        def zrow(i, c):
            for j in range(d // LANES):
                bufA[i, pl.ds(j * LANES, LANES)] = zv
            if with_counts:
                ones[i, pl.ds(0, LANES)] = zv
            return c

        lax.fori_loop(0, CH, zrow, 0)

        def _stripe_copy(src_buf, dst_ref):
            r0 = sid * rpt
            for r in range(rpt // CH):
                pltpu.sync_copy(src_buf, dst_ref.at[pl.ds(r0 + r * CH, CH)])
            rem = rpt % CH
            if rem:
                pltpu.sync_copy(src_buf.at[pl.ds(0, rem)],
                                dst_ref.at[pl.ds(r0 + rpt - rem, rem)])

        _stripe_copy(bufA, acc)
        if with_counts:
            _stripe_copy(ones, cacc)
            ov = jnp.ones((LANES,), jnp.float32)

            def orow(i, c):
                ones[i, pl.ds(0, LANES)] = ov
                return c

            lax.fori_loop(0, CH, orow, 0)

        plsc.subcore_barrier()

        bufs = (bufA, bufB)
        semG = (semA, semB)
        semS = (semSA, semSB)
        pltpu.sync_copy(srcs.at[pl.ds(base, GPC)], sidx.at[0])
        pltpu.sync_copy(dsts.at[pl.ds(base, GPC)], didx.at[0])
        pltpu.async_copy(h.at[sidx.at[0, 0]], bufA, semA)

        def _wait_scatter(b):
            pltpu.make_async_copy(bufs[b], acc.at[didx.at[0, 0]],
                                  semS[b]).wait()

        def _wait_count():
            pltpu.make_async_copy(ones, cacc.at[didx.at[0, 0]],
                                  semC).wait()

        def group(g, carry):
            s = lax.rem(g, 2)
            sn = 1 - s

            for b8 in range(GPC):
                b = b8 % 2
                pltpu.make_async_copy(h.at[sidx.at[s, b8]], bufs[b],
                                      semG[b]).wait()

                if b8 == 2:
                    @pl.when(g + 1 < ngrp)
                    def _():
                        off = base + (g + 1) * GPC
                        pltpu.async_copy(srcs.at[pl.ds(off, GPC)],
                                         sidx.at[sn], semI)
                        pltpu.async_copy(dsts.at[pl.ds(off, GPC)],
                                         didx.at[sn], semI)

                if b8 == 0:
                    @pl.when(g >= 1)
                    def _():
                        _wait_scatter(1 - b)
                else:
                    _wait_scatter(1 - b)

                if b8 + 1 < GPC:
                    pltpu.async_copy(h.at[sidx.at[s, b8 + 1]], bufs[1 - b],
                                     semG[1 - b])
                else:
                    @pl.when(g + 1 < ngrp)
                    def _():
                        pltpu.make_async_copy(srcs.at[pl.ds(base, GPC)],
                                              sidx.at[sn], semI).wait()
                        pltpu.make_async_copy(dsts.at[pl.ds(base, GPC)],
                                              didx.at[sn], semI).wait()
                        pltpu.async_copy(h.at[sidx.at[sn, 0]], bufs[1 - b],
                                         semG[1 - b])

                pltpu.async_copy(bufs[b], acc.at[didx.at[s, b8]],
                                 semS[b], add=True)
                if with_counts:
                    if b8 == 0:
                        @pl.when(g >= 1)
                        def _():
                            _wait_count()
                    else:
                        _wait_count()
                    pltpu.async_copy(ones, cacc.at[didx.at[s, b8]],
                                     semC, add=True)
            return carry

        lax.fori_loop(0, ngrp, group, 0)

        _wait_scatter(1)
        if with_counts:
            _wait_count()

        plsc.subcore_barrier()

        pltpu.sync_copy(acc.at[pl.ds(sid * rpt, rpt)],
                        part.at[cid, pl.ds(sid * rpt, rpt)])
        if with_counts:
            pltpu.sync_copy(cacc.at[pl.ds(sid * rpt, rpt)],
                            cntp.at[cid, pl.ds(sid * rpt, rpt)])

    return pl.kernel(body, out_type=out_type, mesh=mesh,
                     scratch_types=scratch,
                     compiler_params=pltpu.CompilerParams(
                         use_tc_tiling_on_sc=False))


def _pick_br(n):
    for b in (400, 500, 250, 200, 100, 40, 8):
        if n % b == 0:
            return b
    return n


def _combine1(mp, cp, x, wl, bl, wr):
    n, d = x.shape
    br = _pick_br(n)

    def body(mp_ref, cp_ref, x_ref, wl_ref, wr_ref, bl_ref, h_ref, inv_ref):
        cnt = jnp.sum(cp_ref[...], axis=(0, 2)) * (1.0 / CW)
        inv = 1.0 / jnp.maximum(cnt, 1.0)
        inv2 = jnp.broadcast_to(inv[:, None], (br, d))
        m = (mp_ref[0] + mp_ref[1]) * inv2
        o = (jnp.dot(m, wl_ref[...], preferred_element_type=jnp.float32)
             + jnp.dot(x_ref[...], wr_ref[...],
                       preferred_element_type=jnp.float32)
             + bl_ref[...])
        h_ref[...] = jnp.maximum(o, 0.0)
        inv_ref[...] = inv2

    return pl.pallas_call(
        body,
        grid=(n // br,),
        in_specs=[
            pl.BlockSpec((NC, br, d), lambda i: (0, i, 0)),
            pl.BlockSpec((NC, br, CW), lambda i: (0, i, 0)),
            pl.BlockSpec((br, d), lambda i: (i, 0)),
            pl.BlockSpec((d, d), lambda i: (0, 0)),
            pl.BlockSpec((d, d), lambda i: (0, 0)),
            pl.BlockSpec((1, d), lambda i: (0, 0)),
        ],
        out_specs=[
            pl.BlockSpec((br, d), lambda i: (i, 0)),
            pl.BlockSpec((br, d), lambda i: (i, 0)),
        ],
        out_shape=[
            jax.ShapeDtypeStruct((n, d), jnp.float32),
            jax.ShapeDtypeStruct((n, d), jnp.float32),
        ],
    )(mp, cp, x, wl, wr, bl)


def _combine(mp, inv, x, wl, bl, wr, relu):
    n, d = x.shape
    br = _pick_br(n)

    def body(mp_ref, inv_ref, x_ref, wl_ref, wr_ref, bl_ref, h_ref):
        m = (mp_ref[0] + mp_ref[1]) * inv_ref[...]
        o = (jnp.dot(m, wl_ref[...], preferred_element_type=jnp.float32)
             + jnp.dot(x_ref[...], wr_ref[...],
                       preferred_element_type=jnp.float32)
             + bl_ref[...])
        if relu:
            o = jnp.maximum(o, 0.0)
        h_ref[...] = o

    return pl.pallas_call(
        body,
        grid=(n // br,),
        in_specs=[
            pl.BlockSpec((NC, br, d), lambda i: (0, i, 0)),
            pl.BlockSpec((br, d), lambda i: (i, 0)),
            pl.BlockSpec((br, d), lambda i: (i, 0)),
            pl.BlockSpec((d, d), lambda i: (0, 0)),
            pl.BlockSpec((d, d), lambda i: (0, 0)),
            pl.BlockSpec((1, d), lambda i: (0, 0)),
        ],
        out_specs=pl.BlockSpec((br, d), lambda i: (i, 0)),
        out_shape=jax.ShapeDtypeStruct((n, d), jnp.float32),
    )(mp, inv, x, wl, wr, bl)


def kernel(x, edge_index, Wl1, bl1, Wr1, Wl2, bl2, Wr2, Wl3, bl3, Wr3):
    n, d = x.shape
    e = edge_index.shape[1]
    cpt = _round_up(_round_up(e, NW * CH) // (NW * CH), GPC)
    epad = NW * cpt * CH
    nacc = _round_up(n + 1, CH)

    src = edge_index[0].astype(jnp.int32)
    dst = edge_index[1].astype(jnp.int32)
    pad = epad - e
    srcs = jnp.concatenate([src, jnp.zeros((pad,), jnp.int32)])
    dsts = jnp.concatenate([dst, jnp.full((pad,), n, jnp.int32)])
    srcs = srcs.reshape(epad // CH, CH)
    dsts = dsts.reshape(epad // CH, CH)

    agg1 = _make_sc_agg(n, d, cpt, nacc, True)
    agg = _make_sc_agg(n, d, cpt, nacc, False)

    mp1, cp1 = agg1(x, srcs, dsts)
    h1, inv = _combine1(mp1, cp1, x, Wl1, bl1.reshape(1, d), Wr1)
    (mp2,) = agg(h1, srcs, dsts)
    h2 = _combine(mp2, inv, h1, Wl2, bl2.reshape(1, d), Wr2, relu=True)
    (mp3,) = agg(h2, srcs, dsts)
    h3 = _combine(mp3, inv, h2, Wl3, bl3.reshape(1, d), Wr3, relu=False)
    return h3
